# initial kernel scaffold (unmeasured)
import jax
import jax.numpy as jnp
from jax import lax
from jax.experimental import pallas as pl
from jax.experimental.pallas import tpu as pltpu

N_DEV = 8
N_ROUNDS = 3
N_LAYERS = 3
N_SLOTS = N_LAYERS * N_ROUNDS


def kernel(
    x,
    Win0,
    Wout0,
    Win1,
    Wout1,
    Win2,
    Wout2,
):
    b, d_shard = x.shape
    _, h_dim = Win0.shape

    def body(
        x_ref,
        win0_ref,
        wout0_ref,
        win1_ref,
        wout1_ref,
        win2_ref,
        wout2_ref,
        out_ref,
        acc_ref,
        recv_ref,
        send_sems,
        recv_sems,
    ):
        my = lax.axis_index("i")

        barrier_sem = pltpu.get_barrier_semaphore()
        for r in range(N_ROUNDS):
            partner = my ^ (1 << r)
            pl.semaphore_signal(
                barrier_sem,
                inc=1,
                device_id=(partner,),
                device_id_type=pl.DeviceIdType.MESH,
            )
        pl.semaphore_wait(barrier_sem, N_ROUNDS)

        wins = [win0_ref, win1_ref, win2_ref]
        wouts = [wout0_ref, wout1_ref, wout2_ref]

        x_cur = x_ref[:, :]
        for l in range(N_LAYERS):
            acc_ref[:, :] = jnp.dot(
                x_cur, wins[l][:, :], preferred_element_type=jnp.float32
            )
            for r in range(N_ROUNDS):
                k = l * N_ROUNDS + r
                partner = my ^ (1 << r)
                rdma = pltpu.make_async_remote_copy(
                    src_ref=acc_ref,
                    dst_ref=recv_ref.at[k],
                    send_sem=send_sems.at[k],
                    recv_sem=recv_sems.at[k],
                    device_id=(partner,),
                    device_id_type=pl.DeviceIdType.MESH,
                )
                rdma.start()
                rdma.wait()
                acc_ref[:, :] = acc_ref[:, :] + recv_ref[k]
            h = jnp.maximum(acc_ref[:, :], 0.0)
            x_cur = jnp.dot(
                h, wouts[l][:, :], preferred_element_type=jnp.float32
            )
        out_ref[:, :] = x_cur

    return pl.pallas_call(
        body,
        out_shape=jax.ShapeDtypeStruct((b, d_shard), jnp.float32),
        in_specs=[pl.BlockSpec(memory_space=pltpu.VMEM)] * 7,
        out_specs=pl.BlockSpec(memory_space=pltpu.VMEM),
        scratch_shapes=[
            pltpu.VMEM((b, h_dim), jnp.float32),
            pltpu.VMEM((N_SLOTS, b, h_dim), jnp.float32),
            pltpu.SemaphoreType.DMA((N_SLOTS,)),
            pltpu.SemaphoreType.DMA((N_SLOTS,)),
        ],
        compiler_params=pltpu.CompilerParams(collective_id=0),
    )(x, Win0, Wout0, Win1, Wout1, Win2, Wout2)


# baseline (device time: 33507 ns/iter reference)
import jax
import jax.numpy as jnp
from jax import lax
from jax.experimental import pallas as pl
from jax.experimental.pallas import tpu as pltpu

N_DEV = 8
N_ROUNDS = 3
N_LAYERS = 3
N_HALVES = 2
N_SLOTS = N_LAYERS * N_ROUNDS * N_HALVES

MASKS_A = (1, 3, 4)
MASKS_B = (4, 1, 3)
BARRIER_MASKS = (1, 3, 4)


def kernel(
    x,
    Win0,
    Wout0,
    Win1,
    Wout1,
    Win2,
    Wout2,
):
    b, d_shard = x.shape
    _, h_dim = Win0.shape
    half = h_dim // 2

    def body(
        x_ref,
        win0_ref,
        wout0_ref,
        win1_ref,
        wout1_ref,
        win2_ref,
        wout2_ref,
        out_ref,
        acc_a_ref,
        acc_b_ref,
        recv_ref,
        send_sems,
        recv_sems,
    ):
        my = lax.axis_index("i")

        barrier_sem = pltpu.get_barrier_semaphore()
        for mask in BARRIER_MASKS:
            pl.semaphore_signal(
                barrier_sem,
                inc=1,
                device_id=(my ^ mask,),
                device_id_type=pl.DeviceIdType.MESH,
            )
        pl.semaphore_wait(barrier_sem, len(BARRIER_MASKS))

        wins = [win0_ref, win1_ref, win2_ref]
        wouts = [wout0_ref, wout1_ref, wout2_ref]

        x_cur = x_ref[:, :]
        for l in range(N_LAYERS):
            def mk(r, half_idx, acc_ref):
                k = (l * N_ROUNDS + r) * N_HALVES + half_idx
                mask = (MASKS_A, MASKS_B)[half_idx][r]
                return pltpu.make_async_remote_copy(
                    src_ref=acc_ref,
                    dst_ref=recv_ref.at[k],
                    send_sem=send_sems.at[k],
                    recv_sem=recv_sems.at[k],
                    device_id=(my ^ mask,),
                    device_id_type=pl.DeviceIdType.MESH,
                ), k

            rd_a = [mk(r, 0, acc_a_ref) for r in range(N_ROUNDS)]
            rd_b = [mk(r, 1, acc_b_ref) for r in range(N_ROUNDS)]

            acc_a_ref[:, :] = jnp.dot(
                x_cur, wins[l][:, pl.ds(0, half)],
                preferred_element_type=jnp.float32,
            )
            rd_a[0][0].start()
            acc_b_ref[:, :] = jnp.dot(
                x_cur, wins[l][:, pl.ds(half, half)],
                preferred_element_type=jnp.float32,
            )
            rd_b[0][0].start()

            for r in range(N_ROUNDS - 1):
                rdma, k = rd_a[r]
                rdma.wait()
                acc_a_ref[:, :] = acc_a_ref[:, :] + recv_ref[k]
                rd_a[r + 1][0].start()
                rdma, k = rd_b[r]
                rdma.wait()
                acc_b_ref[:, :] = acc_b_ref[:, :] + recv_ref[k]
                rd_b[r + 1][0].start()

            rdma, k = rd_a[N_ROUNDS - 1]
            rdma.wait()
            h_a = jnp.maximum(acc_a_ref[:, :] + recv_ref[k], 0.0)
            y_a = jnp.dot(
                h_a, wouts[l][pl.ds(0, half), :],
                preferred_element_type=jnp.float32,
            )
            rdma, k = rd_b[N_ROUNDS - 1]
            rdma.wait()
            h_b = jnp.maximum(acc_b_ref[:, :] + recv_ref[k], 0.0)
            x_cur = y_a + jnp.dot(
                h_b, wouts[l][pl.ds(half, half), :],
                preferred_element_type=jnp.float32,
            )
        out_ref[:, :] = x_cur

    return pl.pallas_call(
        body,
        out_shape=jax.ShapeDtypeStruct((b, d_shard), jnp.float32),
        in_specs=[pl.BlockSpec(memory_space=pltpu.VMEM)] * 7,
        out_specs=pl.BlockSpec(memory_space=pltpu.VMEM),
        scratch_shapes=[
            pltpu.VMEM((b, half), jnp.float32),
            pltpu.VMEM((b, half), jnp.float32),
            pltpu.VMEM((N_SLOTS, b, half), jnp.float32),
            pltpu.SemaphoreType.DMA((N_SLOTS,)),
            pltpu.SemaphoreType.DMA((N_SLOTS,)),
        ],
        compiler_params=pltpu.CompilerParams(collective_id=0),
    )(x, Win0, Wout0, Win1, Wout1, Win2, Wout2)


# device time: 30437 ns/iter; 1.1009x vs baseline; 1.1009x over previous
import jax
import jax.numpy as jnp
from jax import lax
from jax.experimental import pallas as pl
from jax.experimental.pallas import tpu as pltpu

N_DEV = 8
N_ROUNDS = 3
N_LAYERS = 3
N_HALVES = 2
N_SLOTS = N_LAYERS * N_ROUNDS * N_HALVES

MASKS_A = (1, 3, 4)
MASKS_B = (4, 1, 3)
BARRIER_MASKS = (1, 3, 4)


def kernel(
    x,
    Win0,
    Wout0,
    Win1,
    Wout1,
    Win2,
    Wout2,
):
    b, d_shard = x.shape
    _, h_dim = Win0.shape
    half = h_dim // 2

    def body(
        x_ref,
        win0_ref,
        wout0_ref,
        win1_ref,
        wout1_ref,
        win2_ref,
        wout2_ref,
        out_ref,
        acc_a_ref,
        acc_b_ref,
        recv_ref,
        send_sems,
        recv_sems,
    ):
        my = lax.axis_index("i")

        barrier_sem = pltpu.get_barrier_semaphore()
        for mask in BARRIER_MASKS:
            pl.semaphore_signal(
                barrier_sem,
                inc=1,
                device_id=(my ^ mask,),
                device_id_type=pl.DeviceIdType.MESH,
            )
        pl.semaphore_wait(barrier_sem, len(BARRIER_MASKS))

        wins = [win0_ref, win1_ref, win2_ref]
        wouts = [wout0_ref, wout1_ref, wout2_ref]

        x_cur = x_ref[:, :]
        for l in range(N_LAYERS):
            def mk(r, half_idx, acc_ref):
                k = (l * N_ROUNDS + r) * N_HALVES + half_idx
                mask = (MASKS_A, MASKS_B)[half_idx][r]
                return pltpu.make_async_remote_copy(
                    src_ref=acc_ref,
                    dst_ref=recv_ref.at[k],
                    send_sem=send_sems.at[k],
                    recv_sem=recv_sems.at[k],
                    device_id=(my ^ mask,),
                    device_id_type=pl.DeviceIdType.MESH,
                ), k

            rd_a = [mk(r, 0, acc_a_ref) for r in range(N_ROUNDS)]
            rd_b = [mk(r, 1, acc_b_ref) for r in range(N_ROUNDS)]

            acc_a_ref[:, :] = jnp.dot(
                x_cur, wins[l][:, pl.ds(0, half)],
                preferred_element_type=jnp.float32,
            ).astype(jnp.bfloat16)
            rd_a[0][0].start()
            acc_b_ref[:, :] = jnp.dot(
                x_cur, wins[l][:, pl.ds(half, half)],
                preferred_element_type=jnp.float32,
            ).astype(jnp.bfloat16)
            rd_b[0][0].start()

            for r in range(N_ROUNDS - 1):
                rdma, k = rd_a[r]
                rdma.wait()
                acc_a_ref[:, :] = acc_a_ref[:, :] + recv_ref[k]
                rd_a[r + 1][0].start()
                rdma, k = rd_b[r]
                rdma.wait()
                acc_b_ref[:, :] = acc_b_ref[:, :] + recv_ref[k]
                rd_b[r + 1][0].start()

            rdma, k = rd_a[N_ROUNDS - 1]
            rdma.wait()
            h_a = jnp.maximum(acc_a_ref[:, :] + recv_ref[k], 0.0)
            y_a = jnp.dot(
                h_a, wouts[l][pl.ds(0, half), :].astype(jnp.bfloat16),
                preferred_element_type=jnp.float32,
            )
            rdma, k = rd_b[N_ROUNDS - 1]
            rdma.wait()
            h_b = jnp.maximum(acc_b_ref[:, :] + recv_ref[k], 0.0)
            x_cur = y_a + jnp.dot(
                h_b, wouts[l][pl.ds(half, half), :].astype(jnp.bfloat16),
                preferred_element_type=jnp.float32,
            )
        out_ref[:, :] = x_cur

    return pl.pallas_call(
        body,
        out_shape=jax.ShapeDtypeStruct((b, d_shard), jnp.float32),
        in_specs=[pl.BlockSpec(memory_space=pltpu.VMEM)] * 7,
        out_specs=pl.BlockSpec(memory_space=pltpu.VMEM),
        scratch_shapes=[
            pltpu.VMEM((b, half), jnp.bfloat16),
            pltpu.VMEM((b, half), jnp.bfloat16),
            pltpu.VMEM((N_SLOTS, b, half), jnp.bfloat16),
            pltpu.SemaphoreType.DMA((N_SLOTS,)),
            pltpu.SemaphoreType.DMA((N_SLOTS,)),
        ],
        compiler_params=pltpu.CompilerParams(collective_id=0),
    )(x, Win0, Wout0, Win1, Wout1, Win2, Wout2)


# device time: 28772 ns/iter; 1.1646x vs baseline; 1.0579x over previous
import jax
import jax.numpy as jnp
from jax import lax
from jax.experimental import pallas as pl
from jax.experimental.pallas import tpu as pltpu

N_DEV = 8
N_LAYERS = 3
N_PEERS = N_DEV - 1
N_SLOTS = N_LAYERS * N_PEERS

WAIT_ORDER = (1, 3, 4, 2, 5, 6, 7)


def kernel(
    x,
    Win0,
    Wout0,
    Win1,
    Wout1,
    Win2,
    Wout2,
):
    b, d_shard = x.shape
    _, h_dim = Win0.shape

    def body(
        x_ref,
        win0_ref,
        wout0_ref,
        win1_ref,
        wout1_ref,
        win2_ref,
        wout2_ref,
        out_ref,
        acc_ref,
        recv_ref,
        send_sems,
        recv_sems,
    ):
        my = lax.axis_index("i")

        barrier_sem = pltpu.get_barrier_semaphore()
        for mask in WAIT_ORDER:
            pl.semaphore_signal(
                barrier_sem,
                inc=1,
                device_id=(my ^ mask,),
                device_id_type=pl.DeviceIdType.MESH,
            )
        pl.semaphore_wait(barrier_sem, N_PEERS)

        wins = [win0_ref, win1_ref, win2_ref]
        wouts = [wout0_ref, wout1_ref, wout2_ref]

        x_cur = x_ref[:, :]
        for l in range(N_LAYERS):
            def mk(mask):
                k = l * N_PEERS + (mask - 1)
                return pltpu.make_async_remote_copy(
                    src_ref=acc_ref,
                    dst_ref=recv_ref.at[k],
                    send_sem=send_sems.at[k],
                    recv_sem=recv_sems.at[k],
                    device_id=(my ^ mask,),
                    device_id_type=pl.DeviceIdType.MESH,
                ), k

            acc_ref[:, :] = jnp.dot(
                x_cur, wins[l][:, :], preferred_element_type=jnp.float32
            ).astype(jnp.bfloat16)

            rdmas = [mk(mask) for mask in WAIT_ORDER]
            for rdma, _ in rdmas:
                rdma.start()

            total = acc_ref[:, :].astype(jnp.float32)
            for rdma, k in rdmas:
                rdma.wait()
                total = total + recv_ref[k].astype(jnp.float32)

            h = jnp.maximum(total, 0.0).astype(jnp.bfloat16)
            x_cur = jnp.dot(
                h, wouts[l][:, :].astype(jnp.bfloat16),
                preferred_element_type=jnp.float32,
            )
        out_ref[:, :] = x_cur

    return pl.pallas_call(
        body,
        out_shape=jax.ShapeDtypeStruct((b, d_shard), jnp.float32),
        in_specs=[pl.BlockSpec(memory_space=pltpu.VMEM)] * 7,
        out_specs=pl.BlockSpec(memory_space=pltpu.VMEM),
        scratch_shapes=[
            pltpu.VMEM((b, h_dim), jnp.bfloat16),
            pltpu.VMEM((N_SLOTS, b, h_dim), jnp.bfloat16),
            pltpu.SemaphoreType.DMA((N_SLOTS,)),
            pltpu.SemaphoreType.DMA((N_SLOTS,)),
        ],
        compiler_params=pltpu.CompilerParams(collective_id=0),
    )(x, Win0, Wout0, Win1, Wout1, Win2, Wout2)


# device time: 28657 ns/iter; 1.1692x vs baseline; 1.0040x over previous
import jax
import jax.numpy as jnp
from jax import lax
from jax.experimental import pallas as pl
from jax.experimental.pallas import tpu as pltpu

N_DEV = 8
N_LAYERS = 3
N_PEERS = N_DEV - 1
N_CHUNKS = 2
N_SLOTS = N_LAYERS * N_CHUNKS * N_PEERS

WAIT_ORDER = (1, 3, 4, 2, 5, 6, 7)


def kernel(
    x,
    Win0,
    Wout0,
    Win1,
    Wout1,
    Win2,
    Wout2,
):
    b, d_shard = x.shape
    _, h_dim = Win0.shape
    half = h_dim // N_CHUNKS

    def body(
        x_ref,
        win0_ref,
        wout0_ref,
        win1_ref,
        wout1_ref,
        win2_ref,
        wout2_ref,
        out_ref,
        acc0_ref,
        acc1_ref,
        recv_ref,
        send_sems,
        recv_sems,
    ):
        my = lax.axis_index("i")

        barrier_sem = pltpu.get_barrier_semaphore()
        for mask in WAIT_ORDER:
            pl.semaphore_signal(
                barrier_sem,
                inc=1,
                device_id=(my ^ mask,),
                device_id_type=pl.DeviceIdType.MESH,
            )
        pl.semaphore_wait(barrier_sem, N_PEERS)

        wins = [win0_ref, win1_ref, win2_ref]
        wouts = [wout0_ref, wout1_ref, wout2_ref]
        accs = [acc0_ref, acc1_ref]

        x_cur = x_ref[:, :].astype(jnp.bfloat16)
        for l in range(N_LAYERS):
            def mk(c, mask):
                k = (l * N_CHUNKS + c) * N_PEERS + (mask - 1)
                return pltpu.make_async_remote_copy(
                    src_ref=accs[c],
                    dst_ref=recv_ref.at[k],
                    send_sem=send_sems.at[k],
                    recv_sem=recv_sems.at[k],
                    device_id=(my ^ mask,),
                    device_id_type=pl.DeviceIdType.MESH,
                ), k

            rdmas = [[mk(c, mask) for mask in WAIT_ORDER] for c in range(N_CHUNKS)]

            for c in range(N_CHUNKS):
                accs[c][:, :] = jnp.dot(
                    x_cur,
                    wins[l][:, pl.ds(c * half, half)].astype(jnp.bfloat16),
                    preferred_element_type=jnp.float32,
                ).astype(jnp.bfloat16)
                for rdma, _ in rdmas[c]:
                    rdma.start()

            y = None
            for c in range(N_CHUNKS):
                total = accs[c][:, :].astype(jnp.float32)
                for rdma, k in rdmas[c]:
                    rdma.wait()
                    total = total + recv_ref[k].astype(jnp.float32)
                h = jnp.maximum(total, 0.0).astype(jnp.bfloat16)
                yc = jnp.dot(
                    h,
                    wouts[l][pl.ds(c * half, half), :].astype(jnp.bfloat16),
                    preferred_element_type=jnp.float32,
                )
                y = yc if y is None else y + yc
            x_cur = y.astype(jnp.bfloat16)
        out_ref[:, :] = y

    return pl.pallas_call(
        body,
        out_shape=jax.ShapeDtypeStruct((b, d_shard), jnp.float32),
        in_specs=[pl.BlockSpec(memory_space=pltpu.VMEM)] * 7,
        out_specs=pl.BlockSpec(memory_space=pltpu.VMEM),
        scratch_shapes=[
            pltpu.VMEM((b, half), jnp.bfloat16),
            pltpu.VMEM((b, half), jnp.bfloat16),
            pltpu.VMEM((N_SLOTS, b, half), jnp.bfloat16),
            pltpu.SemaphoreType.DMA((N_SLOTS,)),
            pltpu.SemaphoreType.DMA((N_SLOTS,)),
        ],
        compiler_params=pltpu.CompilerParams(collective_id=0),
    )(x, Win0, Wout0, Win1, Wout1, Win2, Wout2)
